# baseline (device time: 127378 ns/iter reference)
import jax
import jax.numpy as jnp
from jax import lax
from jax.experimental import pallas as pl
from jax.experimental.pallas import tpu as pltpu

N_DEV = 4
N_BLK = 8
SQ = 2048
D_MODEL = 1024
HQ_LOCAL = 8
DH = 128
WINDOW = 128
SCALE = 0.08838834764831843
QBLK = SQ // N_BLK
KW = QBLK + 2 * WINDOW
D_LOCAL = HQ_LOCAL * DH
N_RS = 2 * (N_DEV - 1)


def _k0_of(r0):
    return pl.multiple_of(jnp.clip(r0 - WINDOW, 0, SQ - KW), 128)


def _slot_of(s):
    return s if s < 3 else s - 1


def _body(x_hbm, wq_hbm, k_hbm, v_hbm, wo_hbm, out_ref,
          wf, wq_b, wo_b, ctx_ref, xf, kf, vf, sbuf, rbuf, gbuf,
          w_sem, x_sems, k_sems, v_sems, rs_send, rs_recv, ag_send, ag_recv):
    my = lax.axis_index("i")

    barrier = pltpu.get_barrier_semaphore()
    for rel in range(1, N_DEV):
        pl.semaphore_signal(barrier, inc=1, device_id=((my + rel) % N_DEV,),
                            device_id_type=pl.DeviceIdType.MESH)
    pl.semaphore_wait(barrier, N_DEV - 1)

    def start_fetch(slot, e):
        r0 = e * QBLK
        k0 = _k0_of(r0)
        cps = [pltpu.make_async_copy(
            x_hbm.at[0, pl.ds(r0, QBLK), :], xf.at[slot], x_sems.at[slot])]
        for h in range(HQ_LOCAL):
            cps.append(pltpu.make_async_copy(
                k_hbm.at[0, pl.ds(k0, KW), h, :], kf.at[slot, h],
                k_sems.at[slot, h]))
            cps.append(pltpu.make_async_copy(
                v_hbm.at[0, pl.ds(k0, KW), h, :], vf.at[slot, h],
                v_sems.at[slot, h]))
        for cp in cps:
            cp.start()
        return cps

    fetches = {0: start_fetch(0, (my + 1) % N_BLK)}
    col0 = my * D_LOCAL
    cp_wq = pltpu.make_async_copy(
        wq_hbm.at[:, pl.ds(col0, D_LOCAL)], wf, w_sem)
    cp_wq.start()
    cp_wq.wait()
    wq_b[...] = (wf[...] * SCALE).astype(jnp.bfloat16)
    cp_wo = pltpu.make_async_copy(
        wo_hbm.at[pl.ds(col0, D_LOCAL), :], wf, w_sem)
    cp_wo.start()
    cp_wo.wait()
    wo_b[...] = wf[...].astype(jnp.bfloat16)

    def compute_block(s, e):
        slot = s % 2
        r0 = e * QBLK
        k0 = _k0_of(r0)
        for cp in fetches.pop(slot):
            cp.wait()
        if s < N_BLK - 1:
            fetches[(s + 1) % 2] = start_fetch((s + 1) % 2,
                                               (my + 2 + s) % N_BLK)
        qb = jnp.dot(xf[slot].astype(jnp.bfloat16), wq_b[...],
                     preferred_element_type=jnp.float32)
        qb = qb.astype(jnp.bfloat16)
        ri = lax.broadcasted_iota(jnp.int32, (QBLK, KW), 0) + r0
        ci = lax.broadcasted_iota(jnp.int32, (QBLK, KW), 1) + k0
        maskf = (jnp.abs(ri - ci) <= WINDOW).astype(jnp.float32)
        for h in range(HQ_LOCAL):
            hs = slice(h * DH, (h + 1) * DH)
            kh = kf[slot, h].astype(jnp.bfloat16)
            vh = vf[slot, h].astype(jnp.bfloat16)
            sc = lax.dot_general(qb[:, hs], kh,
                                 (((1,), (1,)), ((), ())),
                                 preferred_element_type=jnp.float32)
            p = jnp.exp(sc) * maskf
            rnorm = 1.0 / jnp.sum(p, axis=1, keepdims=True)
            ctx = jnp.dot(p.astype(jnp.bfloat16), vh,
                          preferred_element_type=jnp.float32)
            ctx_ref[:, hs] = (ctx * rnorm).astype(jnp.bfloat16)
        return jnp.dot(ctx_ref[...], wo_b[...],
                       preferred_element_type=jnp.float32)

    def reduce_and_broadcast(e, first):
        base = 0 if first else 3
        acc = compute_block(3 if first else 7, e)
        for j in range(base, base + 3):
            recv = pltpu.make_async_remote_copy(
                src_ref=rbuf.at[j], dst_ref=rbuf.at[j],
                send_sem=rs_send.at[j], recv_sem=rs_recv.at[j],
                device_id=(my,), device_id_type=pl.DeviceIdType.MESH,
            )
            recv.wait_recv()
            acc = acc + rbuf[j].astype(jnp.float32)
        out_ref[0, pl.ds(e * QBLK, QBLK), :] = acc
        gbuf[pl.ds(e, 1), :, :] = acc.astype(jnp.bfloat16).reshape(
            1, QBLK, D_MODEL)
        rdmas = []
        for rel in range(1, N_DEV):
            c = (my + rel) % N_DEV
            rdma = pltpu.make_async_remote_copy(
                src_ref=gbuf.at[e],
                dst_ref=gbuf.at[e],
                send_sem=ag_send.at[base + rel - 1],
                recv_sem=ag_recv.at[base + N_DEV - 1 - rel],
                device_id=(c,),
                device_id_type=pl.DeviceIdType.MESH,
            )
            rdma.start()
            rdmas.append(rdma)
        return rdmas

    def send_partial(s, e, part):
        slot = _slot_of(s)
        o = e % N_DEV
        cls = (e < N_DEV).astype(jnp.int32)
        rslot = cls * 3 + (my - o) % N_DEV - 1
        sbuf[slot] = part.astype(jnp.bfloat16)
        rdma = pltpu.make_async_remote_copy(
            src_ref=sbuf.at[slot],
            dst_ref=rbuf.at[rslot],
            send_sem=rs_send.at[slot],
            recv_sem=rs_recv.at[rslot],
            device_id=(o,),
            device_id_type=pl.DeviceIdType.MESH,
        )
        rdma.start()
        return rdma

    all_rdmas = []
    for s in (0, 1, 2):
        e = (my + 1 + s) % N_BLK
        all_rdmas.append(send_partial(s, e, compute_block(s, e)))

    all_rdmas += reduce_and_broadcast((my + 4) % N_BLK, first=True)

    for s in (4, 5, 6):
        e = (my + 1 + s) % N_BLK
        all_rdmas.append(send_partial(s, e, compute_block(s, e)))

    all_rdmas += reduce_and_broadcast(my, first=False)

    for j in range(N_RS):
        recv = pltpu.make_async_remote_copy(
            src_ref=gbuf.at[j], dst_ref=gbuf.at[j],
            send_sem=ag_send.at[j], recv_sem=ag_recv.at[j],
            device_id=(my,), device_id_type=pl.DeviceIdType.MESH,
        )
        recv.wait_recv()
    for rel in range(1, N_DEV):
        for base in (0, 4):
            e = (my + rel) % N_DEV + base
            chunk = gbuf[pl.ds(e, 1), :, :].reshape(QBLK, D_MODEL)
            out_ref[0, pl.ds(e * QBLK, QBLK), :] = chunk.astype(jnp.float32)

    for rdma in all_rdmas:
        rdma.wait_send()


def kernel(x, Wq, K_ext, V_ext, Wo):
    out = pl.pallas_call(
        _body,
        out_shape=jax.ShapeDtypeStruct((1, SQ, D_MODEL), jnp.float32),
        in_specs=[pl.BlockSpec(memory_space=pl.ANY)] * 5,
        out_specs=pl.BlockSpec(memory_space=pltpu.VMEM),
        scratch_shapes=[
            pltpu.VMEM((D_MODEL, D_LOCAL), jnp.float32),
            pltpu.VMEM((D_MODEL, D_LOCAL), jnp.bfloat16),
            pltpu.VMEM((D_LOCAL, D_MODEL), jnp.bfloat16),
            pltpu.VMEM((QBLK, HQ_LOCAL * DH), jnp.bfloat16),
            pltpu.VMEM((2, QBLK, D_MODEL), jnp.float32),
            pltpu.VMEM((2, HQ_LOCAL, KW, DH), jnp.float32),
            pltpu.VMEM((2, HQ_LOCAL, KW, DH), jnp.float32),
            pltpu.VMEM((N_RS, QBLK, D_MODEL), jnp.bfloat16),
            pltpu.VMEM((N_RS, QBLK, D_MODEL), jnp.bfloat16),
            pltpu.VMEM((N_BLK, QBLK, D_MODEL), jnp.bfloat16),
            pltpu.SemaphoreType.DMA,
            pltpu.SemaphoreType.DMA((2,)),
            pltpu.SemaphoreType.DMA((2, HQ_LOCAL)),
            pltpu.SemaphoreType.DMA((2, HQ_LOCAL)),
            pltpu.SemaphoreType.DMA((N_RS,)),
            pltpu.SemaphoreType.DMA((N_RS,)),
            pltpu.SemaphoreType.DMA((N_RS,)),
            pltpu.SemaphoreType.DMA((N_RS,)),
        ],
        compiler_params=pltpu.CompilerParams(
            collective_id=0, vmem_limit_bytes=100 * 1024 * 1024),
    )(x, Wq, K_ext, V_ext, Wo)
    return out


# device time: 80771 ns/iter; 1.5770x vs baseline; 1.5770x over previous
import jax
import jax.numpy as jnp
from jax import lax
from jax.experimental import pallas as pl
from jax.experimental.pallas import tpu as pltpu

N_DEV = 4
N_BLK = 8
SQ = 2048
D_MODEL = 1024
HQ_LOCAL = 8
DH = 128
WINDOW = 128
SCALE = 0.08838834764831843
QBLK = SQ // N_BLK
KW = QBLK + 2 * WINDOW
D_LOCAL = HQ_LOCAL * DH
N_RS = 2 * (N_DEV - 1)


def _k0_of(r0):
    return pl.multiple_of(jnp.clip(r0 - WINDOW, 0, SQ - KW), 128)


def _slot_of(s):
    return s if s < 3 else s - 1


def _body(x_hbm, wq_hbm, k_hbm, v_hbm, wo_hbm, out_ref,
          wf, wq_b, wo_b, ctx_ref, xf, kf, vf, sbuf, rbuf, gbuf,
          w_sem, x_sems, k_sems, v_sems, rs_send, rs_recv, ag_send, ag_recv):
    my = lax.axis_index("i")

    barrier = pltpu.get_barrier_semaphore()
    for rel in range(1, N_DEV):
        pl.semaphore_signal(barrier, inc=1, device_id=((my + rel) % N_DEV,),
                            device_id_type=pl.DeviceIdType.MESH)
    pl.semaphore_wait(barrier, N_DEV - 1)

    def start_fetch(slot, e):
        r0 = e * QBLK
        k0 = _k0_of(r0)
        cps = [pltpu.make_async_copy(
            x_hbm.at[0, pl.ds(r0, QBLK), :], xf.at[slot], x_sems.at[slot])]
        for h in range(HQ_LOCAL):
            cps.append(pltpu.make_async_copy(
                k_hbm.at[0, pl.ds(k0, KW), h, :], kf.at[slot, h],
                k_sems.at[slot, h]))
            cps.append(pltpu.make_async_copy(
                v_hbm.at[0, pl.ds(k0, KW), h, :], vf.at[slot, h],
                v_sems.at[slot, h]))
        for cp in cps:
            cp.start()
        return cps

    fetches = {0: start_fetch(0, N_DEV + (my + 1) % N_DEV)}
    col0 = my * D_LOCAL
    cp_wq = pltpu.make_async_copy(
        wq_hbm.at[:, pl.ds(col0, D_LOCAL)], wf, w_sem)
    cp_wq.start()
    cp_wq.wait()
    wq_b[...] = (wf[...] * SCALE).astype(jnp.bfloat16)
    cp_wo = pltpu.make_async_copy(
        wo_hbm.at[pl.ds(col0, D_LOCAL), :], wf, w_sem)
    cp_wo.start()
    cp_wo.wait()
    wo_b[...] = wf[...].astype(jnp.bfloat16)

    seq = ([N_DEV + (my + 1 + s) % N_DEV for s in range(3)]
           + [N_DEV + my]
           + [(my + 1 + s) % N_DEV for s in range(3)]
           + [my])

    def compute_block(s, e):
        slot = s % 2
        r0 = e * QBLK
        k0 = _k0_of(r0)
        for cp in fetches.pop(slot):
            cp.wait()
        if s < N_BLK - 1:
            fetches[(s + 1) % 2] = start_fetch((s + 1) % 2, seq[s + 1])
        qb = jnp.dot(xf[slot].astype(jnp.bfloat16), wq_b[...],
                     preferred_element_type=jnp.float32)
        qb = qb.astype(jnp.bfloat16)
        ri = lax.broadcasted_iota(jnp.int32, (QBLK, KW), 0) + r0
        ci = lax.broadcasted_iota(jnp.int32, (QBLK, KW), 1) + k0
        maskf = (jnp.abs(ri - ci) <= WINDOW).astype(jnp.float32)
        for h in range(HQ_LOCAL):
            hs = slice(h * DH, (h + 1) * DH)
            kh = kf[slot, h].astype(jnp.bfloat16)
            vh = vf[slot, h].astype(jnp.bfloat16)
            sc = lax.dot_general(qb[:, hs], kh,
                                 (((1,), (1,)), ((), ())),
                                 preferred_element_type=jnp.float32)
            p = jnp.exp(sc) * maskf
            rnorm = 1.0 / jnp.sum(p, axis=1, keepdims=True)
            ctx = jnp.dot(p.astype(jnp.bfloat16), vh,
                          preferred_element_type=jnp.float32)
            ctx_ref[:, hs] = (ctx * rnorm).astype(jnp.bfloat16)
        return jnp.dot(ctx_ref[...], wo_b[...],
                       preferred_element_type=jnp.float32)

    def reduce_and_broadcast(e, first):
        base = 0 if first else 3
        acc = compute_block(3 if first else 7, e)
        for j in range(base, base + 3):
            recv = pltpu.make_async_remote_copy(
                src_ref=rbuf.at[j], dst_ref=rbuf.at[j],
                send_sem=rs_send.at[j], recv_sem=rs_recv.at[j],
                device_id=(my,), device_id_type=pl.DeviceIdType.MESH,
            )
            recv.wait_recv()
            acc = acc + rbuf[j].astype(jnp.float32)
        out_ref[0, pl.ds(e * QBLK, QBLK), :] = acc
        gbuf[pl.ds(e, 1), :, :] = acc.astype(jnp.bfloat16).reshape(
            1, QBLK, D_MODEL)
        rdmas = []
        for rel in range(1, N_DEV):
            c = (my + rel) % N_DEV
            rdma = pltpu.make_async_remote_copy(
                src_ref=gbuf.at[e],
                dst_ref=gbuf.at[e],
                send_sem=ag_send.at[base + rel - 1],
                recv_sem=ag_recv.at[base + N_DEV - 1 - rel],
                device_id=(c,),
                device_id_type=pl.DeviceIdType.MESH,
            )
            rdma.start()
            rdmas.append(rdma)
        return rdmas

    def send_partial(s, e, part):
        slot = _slot_of(s)
        o = e % N_DEV
        cls = (e < N_DEV).astype(jnp.int32)
        rslot = cls * 3 + (my - o) % N_DEV - 1
        sbuf[slot] = part.astype(jnp.bfloat16)
        rdma = pltpu.make_async_remote_copy(
            src_ref=sbuf.at[slot],
            dst_ref=rbuf.at[rslot],
            send_sem=rs_send.at[slot],
            recv_sem=rs_recv.at[rslot],
            device_id=(o,),
            device_id_type=pl.DeviceIdType.MESH,
        )
        rdma.start()
        return rdma

    all_rdmas = []
    for s in (0, 1, 2):
        all_rdmas.append(send_partial(s, seq[s], compute_block(s, seq[s])))

    all_rdmas += reduce_and_broadcast(seq[3], first=True)

    for s in (4, 5, 6):
        all_rdmas.append(send_partial(s, seq[s], compute_block(s, seq[s])))

    all_rdmas += reduce_and_broadcast(seq[7], first=False)

    for j in range(N_RS):
        recv = pltpu.make_async_remote_copy(
            src_ref=gbuf.at[j], dst_ref=gbuf.at[j],
            send_sem=ag_send.at[j], recv_sem=ag_recv.at[j],
            device_id=(my,), device_id_type=pl.DeviceIdType.MESH,
        )
        recv.wait_recv()
    for rel in range(1, N_DEV):
        for base in (0, 4):
            e = (my + rel) % N_DEV + base
            chunk = gbuf[pl.ds(e, 1), :, :].reshape(QBLK, D_MODEL)
            out_ref[0, pl.ds(e * QBLK, QBLK), :] = chunk.astype(jnp.float32)

    for rdma in all_rdmas:
        rdma.wait_send()


def kernel(x, Wq, K_ext, V_ext, Wo):
    out = pl.pallas_call(
        _body,
        out_shape=jax.ShapeDtypeStruct((1, SQ, D_MODEL), jnp.float32),
        in_specs=[pl.BlockSpec(memory_space=pl.ANY)] * 5,
        out_specs=pl.BlockSpec(memory_space=pltpu.VMEM),
        scratch_shapes=[
            pltpu.VMEM((D_MODEL, D_LOCAL), jnp.float32),
            pltpu.VMEM((D_MODEL, D_LOCAL), jnp.bfloat16),
            pltpu.VMEM((D_LOCAL, D_MODEL), jnp.bfloat16),
            pltpu.VMEM((QBLK, HQ_LOCAL * DH), jnp.bfloat16),
            pltpu.VMEM((2, QBLK, D_MODEL), jnp.float32),
            pltpu.VMEM((2, HQ_LOCAL, KW, DH), jnp.float32),
            pltpu.VMEM((2, HQ_LOCAL, KW, DH), jnp.float32),
            pltpu.VMEM((N_RS, QBLK, D_MODEL), jnp.bfloat16),
            pltpu.VMEM((N_RS, QBLK, D_MODEL), jnp.bfloat16),
            pltpu.VMEM((N_BLK, QBLK, D_MODEL), jnp.bfloat16),
            pltpu.SemaphoreType.DMA,
            pltpu.SemaphoreType.DMA((2,)),
            pltpu.SemaphoreType.DMA((2, HQ_LOCAL)),
            pltpu.SemaphoreType.DMA((2, HQ_LOCAL)),
            pltpu.SemaphoreType.DMA((N_RS,)),
            pltpu.SemaphoreType.DMA((N_RS,)),
            pltpu.SemaphoreType.DMA((N_RS,)),
            pltpu.SemaphoreType.DMA((N_RS,)),
        ],
        compiler_params=pltpu.CompilerParams(
            collective_id=0, vmem_limit_bytes=100 * 1024 * 1024),
    )(x, Wq, K_ext, V_ext, Wo)
    return out
